# baseline (speedup 1.0000x reference)
import functools

import jax
import jax.numpy as jnp
from jax import lax
from jax.experimental import pallas as pl
from jax.experimental.pallas import tpu as pltpu
from jax.experimental.pallas import tpu_sc as plsc

B = 16384
VOCAB = 1000
TAB = 1024
EMB = 128
HID = 8
NIDX = 2 * B

TILE_B = 4096
NC = 2
NS = 16
NW = NC * NS
BPW = B // NW
CH = 128
NCH = BPW // CH

GROWS = NIDX * 16 // 128
GBLK = TILE_B * 16 // 128
NBLK = B // TILE_B


def _table_body(emb_ref, w_ref, t_ref):
    e = emb_ref[...]
    w = w_ref[...]
    pa = lax.dot_general(e, w[:, :EMB], (((1,), (1,)), ((), ())),
                         preferred_element_type=jnp.float32)
    pb = lax.dot_general(e, w[:, EMB:], (((1,), (1,)), ((), ())),
                         preferred_element_type=jnp.float32)
    t_ref[...] = jnp.zeros((TAB, 16), jnp.float32)
    t_ref[:VOCAB, :] = jnp.concatenate([pa, pb], axis=1)


def _sc_gather_body(t_hbm, x0_hbm, x1_hbm, g_hbm, idx0_v, idx1_v, rows_v, sem):
    c = lax.axis_index("c")
    s = lax.axis_index("s")
    wid = s * NC + c
    pltpu.sync_copy(x0_hbm.at[pl.ds(wid * NCH, NCH)], idx0_v)
    pltpu.sync_copy(x1_hbm.at[pl.ds(wid * NCH, NCH)], idx1_v)
    copies = []
    for j in range(NCH):
        copies.append(pltpu.async_copy(
            t_hbm.at[idx0_v.at[j]], rows_v.at[pl.ds(j * CH, CH)], sem))
        copies.append(pltpu.async_copy(
            t_hbm.at[idx1_v.at[j]],
            rows_v.at[pl.ds(BPW + j * CH, CH)], sem))
    for cp in copies:
        cp.wait()
    pltpu.sync_copy(rows_v.at[pl.ds(0, BPW)], g_hbm.at[pl.ds(wid * BPW, BPW)])
    pltpu.sync_copy(rows_v.at[pl.ds(BPW, BPW)],
                    g_hbm.at[pl.ds(B + wid * BPW, BPW)])


_sc_gather = functools.partial(
    pl.kernel,
    out_type=jax.ShapeDtypeStruct((NIDX, 16), jnp.float32),
    mesh=plsc.VectorSubcoreMesh(core_axis_name="c", subcore_axis_name="s"),
    compiler_params=pltpu.CompilerParams(use_tc_tiling_on_sc=False),
    scratch_types=[
        pltpu.VMEM((NCH, CH), jnp.int32),
        pltpu.VMEM((NCH, CH), jnp.int32),
        pltpu.VMEM((2 * BPW, 16), jnp.float32),
        pltpu.SemaphoreType.DMA,
    ],
)(_sc_gather_body)


def _mlp_body(g0_ref, g1_ref, b1_ref, w_ref, b2_ref, out_ref):
    r0 = g0_ref[...]
    r1 = g1_ref[...]
    r1s = jnp.concatenate([r1[:, HID:], r1[:, :HID]], axis=1)
    q = r0 + r1s
    h8 = jnp.concatenate(
        [q[:, 16 * k:16 * k + HID] for k in range(8)], axis=0)
    hpre = h8 + b1_ref[...]
    h = jnp.tanh(hpre).astype(jnp.bfloat16)
    out_ref[...] = (
        lax.dot_general(w_ref[...], h, (((1,), (1,)), ((), ())),
                        preferred_element_type=jnp.float32)
        + b2_ref[...])


def kernel(x, emb, fc1_w, fc1_b, fc2_w, fc2_b):
    table = pl.pallas_call(
        _table_body,
        out_shape=jax.ShapeDtypeStruct((TAB, 16), jnp.float32),
    )(emb, fc1_w)

    x = x.astype(jnp.int32)
    xp = x.reshape(NBLK, 8, GBLK, 2).transpose(0, 2, 1, 3)
    x0 = xp[..., 0].reshape(B // CH, CH)
    x1 = xp[..., 1].reshape(B // CH, CH)
    g = _sc_gather(table, x0, x1).reshape(GROWS, 128)

    w2 = fc2_w.astype(jnp.bfloat16)
    b1 = fc1_b.reshape(1, HID)
    b2 = fc2_b.reshape(VOCAB, 1)
    out_t = pl.pallas_call(
        _mlp_body,
        grid=(NBLK,),
        in_specs=[
            pl.BlockSpec((GBLK, 128), lambda i: (i, 0)),
            pl.BlockSpec((GBLK, 128), lambda i: (i + NBLK, 0)),
            pl.BlockSpec((1, HID), lambda i: (0, 0)),
            pl.BlockSpec((VOCAB, HID), lambda i: (0, 0)),
            pl.BlockSpec((VOCAB, 1), lambda i: (0, 0)),
        ],
        out_specs=pl.BlockSpec((VOCAB, TILE_B), lambda i: (0, i)),
        out_shape=jax.ShapeDtypeStruct((VOCAB, B), jnp.float32),
    )(g, g, b1, w2, b2)
    return out_t.T


# speedup vs baseline: 1.0016x; 1.0016x over previous
import functools

import jax
import jax.numpy as jnp
from jax import lax
from jax.experimental import pallas as pl
from jax.experimental.pallas import tpu as pltpu
from jax.experimental.pallas import tpu_sc as plsc

B = 16384
VOCAB = 1000
TAB = 1024
EMB = 128
HID = 8
NIDX = 2 * B

TILE_B = 4096
NC = 2
NS = 16
NW = NC * NS
BPW = B // NW
CH = 128
NCH = BPW // CH

GROWS = NIDX * 16 // 128
GBLK = TILE_B * 16 // 128
NBLK = B // TILE_B


def _table_body(emb_ref, w_ref, t_ref):
    e = emb_ref[...]
    w = w_ref[...]
    pa = lax.dot_general(e, w[:, :EMB], (((1,), (1,)), ((), ())),
                         preferred_element_type=jnp.float32)
    pb = lax.dot_general(e, w[:, EMB:], (((1,), (1,)), ((), ())),
                         preferred_element_type=jnp.float32)
    t_ref[...] = jnp.zeros((TAB, 16), jnp.float32)
    t_ref[:VOCAB, :] = jnp.concatenate([pa, pb], axis=1)


def _sc_gather_body(t_hbm, x0_hbm, x1_hbm, g_hbm, idx0_v, idx1_v, rows_v, sem):
    c = lax.axis_index("c")
    s = lax.axis_index("s")
    wid = s * NC + c
    pltpu.sync_copy(x0_hbm.at[pl.ds(wid * NCH, NCH)], idx0_v)
    pltpu.sync_copy(x1_hbm.at[pl.ds(wid * NCH, NCH)], idx1_v)
    copies = []
    for j in range(NCH):
        copies.append(pltpu.async_copy(
            t_hbm.at[idx0_v.at[j]], rows_v.at[pl.ds(j * CH, CH)], sem))
        copies.append(pltpu.async_copy(
            t_hbm.at[idx1_v.at[j]],
            rows_v.at[pl.ds(BPW + j * CH, CH)], sem))
    for cp in copies:
        cp.wait()
    pltpu.sync_copy(rows_v.at[pl.ds(0, BPW)], g_hbm.at[pl.ds(wid * BPW, BPW)])
    pltpu.sync_copy(rows_v.at[pl.ds(BPW, BPW)],
                    g_hbm.at[pl.ds(B + wid * BPW, BPW)])


_sc_gather = functools.partial(
    pl.kernel,
    out_type=jax.ShapeDtypeStruct((NIDX, 16), jnp.float32),
    mesh=plsc.VectorSubcoreMesh(core_axis_name="c", subcore_axis_name="s"),
    compiler_params=pltpu.CompilerParams(use_tc_tiling_on_sc=False,
                                         skip_device_barrier=True),
    scratch_types=[
        pltpu.VMEM((NCH, CH), jnp.int32),
        pltpu.VMEM((NCH, CH), jnp.int32),
        pltpu.VMEM((2 * BPW, 16), jnp.float32),
        pltpu.SemaphoreType.DMA,
    ],
)(_sc_gather_body)


def _mlp_body(g0_ref, g1_ref, b1_ref, w_ref, b2_ref, out_ref):
    r0 = g0_ref[...]
    r1 = g1_ref[...]
    r1s = jnp.concatenate([r1[:, HID:], r1[:, :HID]], axis=1)
    q = r0 + r1s
    h8 = jnp.concatenate(
        [q[:, 16 * k:16 * k + HID] for k in range(8)], axis=0)
    hpre = h8 + b1_ref[...]
    h = jnp.tanh(hpre).astype(jnp.bfloat16)
    out_ref[...] = (
        lax.dot_general(w_ref[...], h, (((1,), (1,)), ((), ())),
                        preferred_element_type=jnp.float32)
        + b2_ref[...])


def kernel(x, emb, fc1_w, fc1_b, fc2_w, fc2_b):
    table = pl.pallas_call(
        _table_body,
        out_shape=jax.ShapeDtypeStruct((TAB, 16), jnp.float32),
    )(emb, fc1_w)

    x = x.astype(jnp.int32)
    xp = x.reshape(NBLK, 8, GBLK, 2).transpose(0, 2, 1, 3)
    x0 = xp[..., 0].reshape(B // CH, CH)
    x1 = xp[..., 1].reshape(B // CH, CH)
    g = _sc_gather(table, x0, x1).reshape(GROWS, 128)

    w2 = fc2_w.astype(jnp.bfloat16)
    b1 = fc1_b.reshape(1, HID)
    b2 = fc2_b.reshape(VOCAB, 1)
    out_t = pl.pallas_call(
        _mlp_body,
        grid=(NBLK,),
        in_specs=[
            pl.BlockSpec((GBLK, 128), lambda i: (i, 0)),
            pl.BlockSpec((GBLK, 128), lambda i: (i + NBLK, 0)),
            pl.BlockSpec((1, HID), lambda i: (0, 0)),
            pl.BlockSpec((VOCAB, HID), lambda i: (0, 0)),
            pl.BlockSpec((VOCAB, 1), lambda i: (0, 0)),
        ],
        out_specs=pl.BlockSpec((VOCAB, TILE_B), lambda i: (0, i)),
        out_shape=jax.ShapeDtypeStruct((VOCAB, B), jnp.float32),
    )(g, g, b1, w2, b2)
    return out_t.T
